# baseline (speedup 1.0000x reference)
import functools

import jax
import jax.numpy as jnp
from jax import lax
from jax.experimental import pallas as pl
from jax.experimental.pallas import tpu as pltpu
from jax.experimental.pallas import tpu_sc as plsc

N = 100000
D = 128
H = 64
G = 512


_BLK = 2048
_GRID = -(-N // _BLK)
_OROWS = _GRID * _BLK // 128


def _mlp_body(x_ref, w1_ref, b1_ref, w2_ref, b2_ref, o_ref):
    i = pl.program_id(0)
    x = x_ref[...]
    h = jnp.dot(x, w1_ref[...], preferred_element_type=jnp.float32)
    h = h + b1_ref[...]
    h = h * jax.nn.sigmoid(h)
    r = jnp.sum(h * w2_ref[...], axis=1) + b2_ref[0, 0]
    r = r.reshape(_BLK // 128, 128)
    gid = i * _BLK + jax.lax.broadcasted_iota(jnp.int32, (_BLK // 128, 128), 0) * 128 \
        + jax.lax.broadcasted_iota(jnp.int32, (_BLK // 128, 128), 1)
    o_ref[...] = jnp.where(gid < N, r, 0.0)


def _mlp(x, W1, b1, W2, b2):
    return pl.pallas_call(
        _mlp_body,
        grid=(_GRID,),
        in_specs=[
            pl.BlockSpec((_BLK, D), lambda i: (i, 0)),
            pl.BlockSpec((D, H), lambda i: (0, 0)),
            pl.BlockSpec((1, H), lambda i: (0, 0)),
            pl.BlockSpec((1, H), lambda i: (0, 0)),
            pl.BlockSpec((1, 1), lambda i: (0, 0)),
        ],
        out_specs=pl.BlockSpec((_BLK // 128, 128), lambda i: (i, 0)),
        out_shape=jax.ShapeDtypeStruct((_OROWS, 128), jnp.float32),
    )(x, W1, b1.reshape(1, H), W2.reshape(1, H), b2.reshape(1, 1))


_NS = 16
_ROWS = 896
_NPAD = _ROWS * 128
_RPW = _ROWS // _NS
_CHUNK = 14
_SPREAD = 16
_PLANE = G + 16
_ACC = _SPREAD * _PLANE
_GPW = G // _NS


def _segsum_body(vals_hbm, idx_hbm, out_hbm, vals_v, idx_v, stage_v, fold_v,
                 out_v, acc_sp, in_sem, sc_sem):
    s = lax.axis_index("s")
    base = s * _RPW
    vals_cp = pltpu.async_copy(vals_hbm.at[pl.ds(base, _RPW)], vals_v, in_sem)
    idx_cp = pltpu.async_copy(idx_hbm.at[pl.ds(base, _RPW)], idx_v, in_sem)

    @pl.when(s == 0)
    def _():
        for j in range(_ACC // 16):
            stage_v[pl.ds(j * 16, 16)] = jnp.zeros((16,), jnp.float32)
        pltpu.sync_copy(stage_v, acc_sp)

    vals_cp.wait()
    idx_cp.wait()
    plsc.subcore_barrier()

    def chunk(k, carry):
        cps = [
            pltpu.async_copy(
                vals_v.at[k * _CHUNK + j],
                acc_sp.at[idx_v.at[k * _CHUNK + j]],
                sc_sem,
                add=True,
            )
            for j in range(_CHUNK)
        ]
        for cp in cps:
            cp.wait()
        return carry

    lax.fori_loop(0, _RPW // _CHUNK, chunk, 0)
    plsc.subcore_barrier()

    fold_cps = [
        pltpu.async_copy(
            acc_sp.at[pl.ds(j * _PLANE + s * _GPW, _GPW)],
            fold_v.at[pl.ds(j * _GPW, _GPW)],
            in_sem,
        )
        for j in range(_SPREAD)
    ]
    for cp in fold_cps:
        cp.wait()
    for gb in range(_GPW // 16):
        tot = jnp.zeros((16,), jnp.float32)
        for j in range(_SPREAD):
            tot = tot + fold_v[pl.ds(j * _GPW + gb * 16, 16)]
        out_v[pl.ds(gb * 16, 16)] = tot
    pltpu.sync_copy(out_v, out_hbm.at[pl.ds(s * _GPW, _GPW)])


@functools.cache
def _make_segsum():
    return pl.kernel(
        _segsum_body,
        out_type=jax.ShapeDtypeStruct((G,), jnp.float32),
        mesh=plsc.VectorSubcoreMesh(
            core_axis_name="c", subcore_axis_name="s",
            num_cores=1, num_subcores=_NS,
        ),
        scratch_types=[
            pltpu.VMEM((_RPW, 128), jnp.float32),
            pltpu.VMEM((_RPW, 128), jnp.int32),
            pltpu.VMEM((_ACC,), jnp.float32),
            pltpu.VMEM((_SPREAD * _GPW,), jnp.float32),
            pltpu.VMEM((_GPW,), jnp.float32),
            pltpu.VMEM_SHARED((_ACC,), jnp.float32),
            pltpu.SemaphoreType.DMA,
            pltpu.SemaphoreType.DMA,
        ],
    )


def kernel(x_scalar, x_spherical, batch, W1, b1, W2, b2):
    res = _mlp(x_scalar, W1, b1, W2, b2)
    vals = jnp.pad(res, ((0, _ROWS - _OROWS), (0, 0)))
    spread = jnp.arange(N, dtype=jnp.int32) % _SPREAD
    idx = jnp.pad(batch + spread * _PLANE, (0, _NPAD - N))
    pad_pos = jnp.arange(_NPAD, dtype=jnp.int32)
    idx = jnp.where(pad_pos < N, idx, pad_pos % _ACC).reshape(_ROWS, 128)
    out = _make_segsum()(vals, idx)
    return out.reshape(G, 1)


# speedup vs baseline: 1.0100x; 1.0100x over previous
import functools

import jax
import jax.numpy as jnp
from jax import lax
from jax.experimental import pallas as pl
from jax.experimental.pallas import tpu as pltpu
from jax.experimental.pallas import tpu_sc as plsc

N = 100000
D = 128
H = 64
G = 512


_BLK = 2048
_GRID = -(-N // _BLK)
_OROWS = _GRID * _BLK // 128


def _mlp_body(x_ref, w1_ref, b1_ref, w2_ref, b2_ref, o_ref):
    i = pl.program_id(0)
    x = x_ref[...]
    h = jnp.dot(x, w1_ref[...], preferred_element_type=jnp.float32)
    h = h + b1_ref[...]
    h = h * jax.nn.sigmoid(h)
    r = jnp.sum(h * w2_ref[...], axis=1) + b2_ref[0, 0]
    r = r.reshape(_BLK // 128, 128)
    gid = i * _BLK + jax.lax.broadcasted_iota(jnp.int32, (_BLK // 128, 128), 0) * 128 \
        + jax.lax.broadcasted_iota(jnp.int32, (_BLK // 128, 128), 1)
    o_ref[...] = jnp.where(gid < N, r, 0.0)


def _mlp(x, W1, b1, W2, b2):
    return pl.pallas_call(
        _mlp_body,
        grid=(_GRID,),
        in_specs=[
            pl.BlockSpec((_BLK, D), lambda i: (i, 0)),
            pl.BlockSpec((D, H), lambda i: (0, 0)),
            pl.BlockSpec((1, H), lambda i: (0, 0)),
            pl.BlockSpec((1, H), lambda i: (0, 0)),
            pl.BlockSpec((1, 1), lambda i: (0, 0)),
        ],
        out_specs=pl.BlockSpec((_BLK // 128, 128), lambda i: (i, 0)),
        out_shape=jax.ShapeDtypeStruct((_OROWS, 128), jnp.float32),
    )(x, W1, b1.reshape(1, H), W2.reshape(1, H), b2.reshape(1, 1))


_NS = 16
_ROWS = 896
_NPAD = _ROWS * 128
_RPW = _ROWS // _NS
_CHUNK = 28
_SPREAD = 8
_PLANE = G + 16
_ACC = _SPREAD * _PLANE
_GPW = G // _NS


def _segsum_body(vals_hbm, idx_hbm, out_hbm, vals_v, idx_v, stage_v, fold_v,
                 out_v, acc_sp, in_sem, sc_sem):
    s = lax.axis_index("s")
    base = s * _RPW
    vals_cp = pltpu.async_copy(vals_hbm.at[pl.ds(base, _RPW)], vals_v, in_sem)
    idx_cp = pltpu.async_copy(idx_hbm.at[pl.ds(base, _RPW)], idx_v, in_sem)

    @pl.when(s == 0)
    def _():
        for j in range(_ACC // 16):
            stage_v[pl.ds(j * 16, 16)] = jnp.zeros((16,), jnp.float32)
        pltpu.sync_copy(stage_v, acc_sp)

    vals_cp.wait()
    idx_cp.wait()
    plsc.subcore_barrier()

    def chunk(k, carry):
        cps = [
            pltpu.async_copy(
                vals_v.at[k * _CHUNK + j],
                acc_sp.at[idx_v.at[k * _CHUNK + j]],
                sc_sem,
                add=True,
            )
            for j in range(_CHUNK)
        ]
        for cp in cps:
            cp.wait()
        return carry

    lax.fori_loop(0, _RPW // _CHUNK, chunk, 0)
    plsc.subcore_barrier()

    fold_cps = [
        pltpu.async_copy(
            acc_sp.at[pl.ds(j * _PLANE + s * _GPW, _GPW)],
            fold_v.at[pl.ds(j * _GPW, _GPW)],
            in_sem,
        )
        for j in range(_SPREAD)
    ]
    for cp in fold_cps:
        cp.wait()
    for gb in range(_GPW // 16):
        tot = jnp.zeros((16,), jnp.float32)
        for j in range(_SPREAD):
            tot = tot + fold_v[pl.ds(j * _GPW + gb * 16, 16)]
        out_v[pl.ds(gb * 16, 16)] = tot
    pltpu.sync_copy(out_v, out_hbm.at[pl.ds(s * _GPW, _GPW)])


@functools.cache
def _make_segsum():
    return pl.kernel(
        _segsum_body,
        out_type=jax.ShapeDtypeStruct((G,), jnp.float32),
        mesh=plsc.VectorSubcoreMesh(
            core_axis_name="c", subcore_axis_name="s",
            num_cores=1, num_subcores=_NS,
        ),
        scratch_types=[
            pltpu.VMEM((_RPW, 128), jnp.float32),
            pltpu.VMEM((_RPW, 128), jnp.int32),
            pltpu.VMEM((_ACC,), jnp.float32),
            pltpu.VMEM((_SPREAD * _GPW,), jnp.float32),
            pltpu.VMEM((_GPW,), jnp.float32),
            pltpu.VMEM_SHARED((_ACC,), jnp.float32),
            pltpu.SemaphoreType.DMA,
            pltpu.SemaphoreType.DMA,
        ],
    )


def kernel(x_scalar, x_spherical, batch, W1, b1, W2, b2):
    res = _mlp(x_scalar, W1, b1, W2, b2)
    vals = jnp.pad(res, ((0, _ROWS - _OROWS), (0, 0)))
    spread = jnp.arange(N, dtype=jnp.int32) % _SPREAD
    idx = jnp.pad(batch + spread * _PLANE, (0, _NPAD - N))
    pad_pos = jnp.arange(_NPAD, dtype=jnp.int32)
    idx = jnp.where(pad_pos < N, idx, pad_pos % _ACC).reshape(_ROWS, 128)
    out = _make_segsum()(vals, idx)
    return out.reshape(G, 1)


# speedup vs baseline: 1.2346x; 1.2224x over previous
import functools

import jax
import jax.numpy as jnp
from jax import lax
from jax.experimental import pallas as pl
from jax.experimental.pallas import tpu as pltpu
from jax.experimental.pallas import tpu_sc as plsc

N = 100000
D = 128
H = 64
G = 512


_BLK = 4096
_GRID = -(-N // _BLK)
_OROWS = _GRID * _BLK // 128


def _mlp_body(x_ref, w1_ref, b1_ref, w2_ref, b2_ref, o_ref):
    i = pl.program_id(0)
    x = x_ref[...]
    h = jnp.dot(x, w1_ref[...], preferred_element_type=jnp.float32)
    h = h + b1_ref[...]
    h = h * jax.nn.sigmoid(h)
    r = jnp.sum(h * w2_ref[...], axis=1) + b2_ref[0, 0]
    r = r.reshape(_BLK // 128, 128)
    gid = i * _BLK + jax.lax.broadcasted_iota(jnp.int32, (_BLK // 128, 128), 0) * 128 \
        + jax.lax.broadcasted_iota(jnp.int32, (_BLK // 128, 128), 1)
    o_ref[...] = jnp.where(gid < N, r, 0.0)


def _mlp(x, W1, b1, W2, b2):
    return pl.pallas_call(
        _mlp_body,
        grid=(_GRID,),
        in_specs=[
            pl.BlockSpec((_BLK, D), lambda i: (i, 0)),
            pl.BlockSpec((D, H), lambda i: (0, 0)),
            pl.BlockSpec((1, H), lambda i: (0, 0)),
            pl.BlockSpec((1, H), lambda i: (0, 0)),
            pl.BlockSpec((1, 1), lambda i: (0, 0)),
        ],
        out_specs=pl.BlockSpec((_BLK // 128, 128), lambda i: (i, 0)),
        out_shape=jax.ShapeDtypeStruct((_OROWS, 128), jnp.float32),
    )(x, W1, b1.reshape(1, H), W2.reshape(1, H), b2.reshape(1, 1))


_NS = 16
_ROWS = 896
_NPAD = _ROWS * 128
_RPW = _ROWS // _NS
_CHUNK = 28
_SPREAD = 8
_PLANE = G + 16
_ACC = _SPREAD * _PLANE
_GPW = G // _NS


def _segsum_body(vals_hbm, idx_hbm, out_hbm, vals_v, idx_v, stage_v, fold_v,
                 out_v, acc_sp, in_sem, sc_sem):
    s = lax.axis_index("s")
    base = s * _RPW
    vals_cp = pltpu.async_copy(vals_hbm.at[pl.ds(base, _RPW)], vals_v, in_sem)
    idx_cp = pltpu.async_copy(idx_hbm.at[pl.ds(base, _RPW)], idx_v, in_sem)

    @pl.when(s == 0)
    def _():
        for j in range(_ACC // 16):
            stage_v[pl.ds(j * 16, 16)] = jnp.zeros((16,), jnp.float32)
        pltpu.sync_copy(stage_v, acc_sp)

    vals_cp.wait()
    idx_cp.wait()
    plsc.subcore_barrier()

    def chunk(k, carry):
        cps = [
            pltpu.async_copy(
                vals_v.at[k * _CHUNK + j],
                acc_sp.at[idx_v.at[k * _CHUNK + j]],
                sc_sem,
                add=True,
            )
            for j in range(_CHUNK)
        ]
        for cp in cps:
            cp.wait()
        return carry

    lax.fori_loop(0, _RPW // _CHUNK, chunk, 0)
    plsc.subcore_barrier()

    fold_cps = [
        pltpu.async_copy(
            acc_sp.at[pl.ds(j * _PLANE + s * _GPW, _GPW)],
            fold_v.at[pl.ds(j * _GPW, _GPW)],
            in_sem,
        )
        for j in range(_SPREAD)
    ]
    for cp in fold_cps:
        cp.wait()
    for gb in range(_GPW // 16):
        tot = jnp.zeros((16,), jnp.float32)
        for j in range(_SPREAD):
            tot = tot + fold_v[pl.ds(j * _GPW + gb * 16, 16)]
        out_v[pl.ds(gb * 16, 16)] = tot
    pltpu.sync_copy(out_v, out_hbm.at[pl.ds(s * _GPW, _GPW)])


@functools.cache
def _make_segsum():
    return pl.kernel(
        _segsum_body,
        out_type=jax.ShapeDtypeStruct((G,), jnp.float32),
        mesh=plsc.VectorSubcoreMesh(
            core_axis_name="c", subcore_axis_name="s",
            num_cores=1, num_subcores=_NS,
        ),
        scratch_types=[
            pltpu.VMEM((_RPW, 128), jnp.float32),
            pltpu.VMEM((_RPW, 128), jnp.int32),
            pltpu.VMEM((_ACC,), jnp.float32),
            pltpu.VMEM((_SPREAD * _GPW,), jnp.float32),
            pltpu.VMEM((_GPW,), jnp.float32),
            pltpu.VMEM_SHARED((_ACC,), jnp.float32),
            pltpu.SemaphoreType.DMA,
            pltpu.SemaphoreType.DMA,
        ],
    )


def kernel(x_scalar, x_spherical, batch, W1, b1, W2, b2):
    res = _mlp(x_scalar, W1, b1, W2, b2)
    vals = jnp.pad(res, ((0, _ROWS - _OROWS), (0, 0)))
    spread = jnp.arange(N, dtype=jnp.int32) % _SPREAD
    idx = jnp.pad(batch + spread * _PLANE, (0, _NPAD - N))
    pad_pos = jnp.arange(_NPAD, dtype=jnp.int32)
    idx = jnp.where(pad_pos < N, idx, pad_pos % _ACC).reshape(_ROWS, 128)
    out = _make_segsum()(vals, idx)
    return out.reshape(G, 1)


# speedup vs baseline: 1.3709x; 1.1104x over previous
import functools

import jax
import jax.numpy as jnp
from jax import lax
from jax.experimental import pallas as pl
from jax.experimental.pallas import tpu as pltpu
from jax.experimental.pallas import tpu_sc as plsc

N = 100000
D = 128
H = 64
G = 512


_BLK = 8192
_GRID = -(-N // _BLK)
_OROWS = _GRID * _BLK // 128


def _mlp_body(x_ref, w1_ref, b1_ref, w2_ref, b2_ref, o_ref):
    i = pl.program_id(0)
    x = x_ref[...]
    h = jnp.dot(x, w1_ref[...], preferred_element_type=jnp.float32)
    h = h + b1_ref[...]
    h = h * jax.nn.sigmoid(h)
    r = jnp.sum(h * w2_ref[...], axis=1) + b2_ref[0, 0]
    r = r.reshape(_BLK // 128, 128)
    gid = i * _BLK + jax.lax.broadcasted_iota(jnp.int32, (_BLK // 128, 128), 0) * 128 \
        + jax.lax.broadcasted_iota(jnp.int32, (_BLK // 128, 128), 1)
    o_ref[...] = jnp.where(gid < N, r, 0.0)


def _mlp(x, W1, b1, W2, b2):
    return pl.pallas_call(
        _mlp_body,
        grid=(_GRID,),
        in_specs=[
            pl.BlockSpec((_BLK, D), lambda i: (i, 0)),
            pl.BlockSpec((D, H), lambda i: (0, 0)),
            pl.BlockSpec((1, H), lambda i: (0, 0)),
            pl.BlockSpec((1, H), lambda i: (0, 0)),
            pl.BlockSpec((1, 1), lambda i: (0, 0)),
        ],
        out_specs=pl.BlockSpec((_BLK // 128, 128), lambda i: (i, 0)),
        out_shape=jax.ShapeDtypeStruct((_OROWS, 128), jnp.float32),
    )(x, W1, b1.reshape(1, H), W2.reshape(1, H), b2.reshape(1, 1))


_NS = 16
_ROWS = 896
_NPAD = _ROWS * 128
_RPW = _ROWS // _NS
_CHUNK = 28
_SPREAD = 8
_PLANE = G + 16
_ACC = _SPREAD * _PLANE
_GPW = G // _NS


def _segsum_body(vals_hbm, idx_hbm, out_hbm, vals_v, idx_v, stage_v, fold_v,
                 out_v, acc_sp, in_sem, sc_sem):
    s = lax.axis_index("s")
    base = s * _RPW
    vals_cp = pltpu.async_copy(vals_hbm.at[pl.ds(base, _RPW)], vals_v, in_sem)
    idx_cp = pltpu.async_copy(idx_hbm.at[pl.ds(base, _RPW)], idx_v, in_sem)

    @pl.when(s == 0)
    def _():
        for j in range(_ACC // 16):
            stage_v[pl.ds(j * 16, 16)] = jnp.zeros((16,), jnp.float32)
        pltpu.sync_copy(stage_v, acc_sp)

    vals_cp.wait()
    idx_cp.wait()
    plsc.subcore_barrier()

    def chunk(k, carry):
        cps = [
            pltpu.async_copy(
                vals_v.at[k * _CHUNK + j],
                acc_sp.at[idx_v.at[k * _CHUNK + j]],
                sc_sem,
                add=True,
            )
            for j in range(_CHUNK)
        ]
        for cp in cps:
            cp.wait()
        return carry

    lax.fori_loop(0, _RPW // _CHUNK, chunk, 0)
    plsc.subcore_barrier()

    fold_cps = [
        pltpu.async_copy(
            acc_sp.at[pl.ds(j * _PLANE + s * _GPW, _GPW)],
            fold_v.at[pl.ds(j * _GPW, _GPW)],
            in_sem,
        )
        for j in range(_SPREAD)
    ]
    for cp in fold_cps:
        cp.wait()
    for gb in range(_GPW // 16):
        tot = jnp.zeros((16,), jnp.float32)
        for j in range(_SPREAD):
            tot = tot + fold_v[pl.ds(j * _GPW + gb * 16, 16)]
        out_v[pl.ds(gb * 16, 16)] = tot
    pltpu.sync_copy(out_v, out_hbm.at[pl.ds(s * _GPW, _GPW)])


@functools.cache
def _make_segsum():
    return pl.kernel(
        _segsum_body,
        out_type=jax.ShapeDtypeStruct((G,), jnp.float32),
        mesh=plsc.VectorSubcoreMesh(
            core_axis_name="c", subcore_axis_name="s",
            num_cores=1, num_subcores=_NS,
        ),
        scratch_types=[
            pltpu.VMEM((_RPW, 128), jnp.float32),
            pltpu.VMEM((_RPW, 128), jnp.int32),
            pltpu.VMEM((_ACC,), jnp.float32),
            pltpu.VMEM((_SPREAD * _GPW,), jnp.float32),
            pltpu.VMEM((_GPW,), jnp.float32),
            pltpu.VMEM_SHARED((_ACC,), jnp.float32),
            pltpu.SemaphoreType.DMA,
            pltpu.SemaphoreType.DMA,
        ],
    )


def kernel(x_scalar, x_spherical, batch, W1, b1, W2, b2):
    res = _mlp(x_scalar, W1, b1, W2, b2)
    vals = jnp.pad(res, ((0, _ROWS - _OROWS), (0, 0)))
    spread = jnp.arange(N, dtype=jnp.int32) % _SPREAD
    idx = jnp.pad(batch + spread * _PLANE, (0, _NPAD - N))
    pad_pos = jnp.arange(_NPAD, dtype=jnp.int32)
    idx = jnp.where(pad_pos < N, idx, pad_pos % _ACC).reshape(_ROWS, 128)
    out = _make_segsum()(vals, idx)
    return out.reshape(G, 1)


# speedup vs baseline: 1.4345x; 1.0464x over previous
import functools

import jax
import jax.numpy as jnp
from jax import lax
from jax.experimental import pallas as pl
from jax.experimental.pallas import tpu as pltpu
from jax.experimental.pallas import tpu_sc as plsc

N = 100000
D = 128
H = 64
G = 512


_BLK = 16384
_GRID = -(-N // _BLK)
_OROWS = _GRID * _BLK // 128


def _mlp_body(x_ref, w1_ref, b1_ref, w2_ref, b2_ref, o_ref):
    i = pl.program_id(0)
    x = x_ref[...]
    h = jnp.dot(x, w1_ref[...], preferred_element_type=jnp.float32)
    h = h + b1_ref[...]
    h = h * jax.nn.sigmoid(h)
    r = jnp.sum(h * w2_ref[...], axis=1) + b2_ref[0, 0]
    r = r.reshape(_BLK // 128, 128)
    gid = i * _BLK + jax.lax.broadcasted_iota(jnp.int32, (_BLK // 128, 128), 0) * 128 \
        + jax.lax.broadcasted_iota(jnp.int32, (_BLK // 128, 128), 1)
    o_ref[...] = jnp.where(gid < N, r, 0.0)


def _mlp(x, W1, b1, W2, b2):
    return pl.pallas_call(
        _mlp_body,
        grid=(_GRID,),
        in_specs=[
            pl.BlockSpec((_BLK, D), lambda i: (i, 0)),
            pl.BlockSpec((D, H), lambda i: (0, 0)),
            pl.BlockSpec((1, H), lambda i: (0, 0)),
            pl.BlockSpec((1, H), lambda i: (0, 0)),
            pl.BlockSpec((1, 1), lambda i: (0, 0)),
        ],
        out_specs=pl.BlockSpec((_BLK // 128, 128), lambda i: (i, 0)),
        out_shape=jax.ShapeDtypeStruct((_OROWS, 128), jnp.float32),
    )(x, W1, b1.reshape(1, H), W2.reshape(1, H), b2.reshape(1, 1))


_NS = 16
_ROWS = 896
_NPAD = _ROWS * 128
_RPW = _ROWS // _NS
_CHUNK = 28
_SPREAD = 8
_PLANE = G + 16
_ACC = _SPREAD * _PLANE
_GPW = G // _NS


def _segsum_body(vals_hbm, idx_hbm, out_hbm, vals_v, idx_v, stage_v, fold_v,
                 out_v, acc_sp, in_sem, sc_sem):
    s = lax.axis_index("s")
    base = s * _RPW
    vals_cp = pltpu.async_copy(vals_hbm.at[pl.ds(base, _RPW)], vals_v, in_sem)
    idx_cp = pltpu.async_copy(idx_hbm.at[pl.ds(base, _RPW)], idx_v, in_sem)

    @pl.when(s == 0)
    def _():
        for j in range(_ACC // 16):
            stage_v[pl.ds(j * 16, 16)] = jnp.zeros((16,), jnp.float32)
        pltpu.sync_copy(stage_v, acc_sp)

    vals_cp.wait()
    idx_cp.wait()
    plsc.subcore_barrier()

    def chunk(k, carry):
        cps = [
            pltpu.async_copy(
                vals_v.at[k * _CHUNK + j],
                acc_sp.at[idx_v.at[k * _CHUNK + j]],
                sc_sem,
                add=True,
            )
            for j in range(_CHUNK)
        ]
        for cp in cps:
            cp.wait()
        return carry

    lax.fori_loop(0, _RPW // _CHUNK, chunk, 0)
    plsc.subcore_barrier()

    fold_cps = [
        pltpu.async_copy(
            acc_sp.at[pl.ds(j * _PLANE + s * _GPW, _GPW)],
            fold_v.at[pl.ds(j * _GPW, _GPW)],
            in_sem,
        )
        for j in range(_SPREAD)
    ]
    for cp in fold_cps:
        cp.wait()
    for gb in range(_GPW // 16):
        tot = jnp.zeros((16,), jnp.float32)
        for j in range(_SPREAD):
            tot = tot + fold_v[pl.ds(j * _GPW + gb * 16, 16)]
        out_v[pl.ds(gb * 16, 16)] = tot
    pltpu.sync_copy(out_v, out_hbm.at[pl.ds(s * _GPW, _GPW)])


@functools.cache
def _make_segsum():
    return pl.kernel(
        _segsum_body,
        out_type=jax.ShapeDtypeStruct((G,), jnp.float32),
        mesh=plsc.VectorSubcoreMesh(
            core_axis_name="c", subcore_axis_name="s",
            num_cores=1, num_subcores=_NS,
        ),
        scratch_types=[
            pltpu.VMEM((_RPW, 128), jnp.float32),
            pltpu.VMEM((_RPW, 128), jnp.int32),
            pltpu.VMEM((_ACC,), jnp.float32),
            pltpu.VMEM((_SPREAD * _GPW,), jnp.float32),
            pltpu.VMEM((_GPW,), jnp.float32),
            pltpu.VMEM_SHARED((_ACC,), jnp.float32),
            pltpu.SemaphoreType.DMA,
            pltpu.SemaphoreType.DMA,
        ],
    )


def kernel(x_scalar, x_spherical, batch, W1, b1, W2, b2):
    res = _mlp(x_scalar, W1, b1, W2, b2)
    vals = jnp.pad(res, ((0, _ROWS - _OROWS), (0, 0)))
    spread = jnp.arange(N, dtype=jnp.int32) % _SPREAD
    idx = jnp.pad(batch + spread * _PLANE, (0, _NPAD - N))
    pad_pos = jnp.arange(_NPAD, dtype=jnp.int32)
    idx = jnp.where(pad_pos < N, idx, pad_pos % _ACC).reshape(_ROWS, 128)
    out = _make_segsum()(vals, idx)
    return out.reshape(G, 1)


# speedup vs baseline: 1.4624x; 1.0194x over previous
import functools

import jax
import jax.numpy as jnp
from jax import lax
from jax.experimental import pallas as pl
from jax.experimental.pallas import tpu as pltpu
from jax.experimental.pallas import tpu_sc as plsc

N = 100000
D = 128
H = 64
G = 512


_BLK = 14336
_GRID = -(-N // _BLK)
_OROWS = _GRID * _BLK // 128


def _mlp_body(x_ref, w1_ref, b1_ref, w2_ref, b2_ref, o_ref):
    i = pl.program_id(0)
    x = x_ref[...]
    h = jnp.dot(x, w1_ref[...], preferred_element_type=jnp.float32)
    h = h + b1_ref[...]
    h = h * jax.nn.sigmoid(h)
    r = jnp.sum(h * w2_ref[...], axis=1) + b2_ref[0, 0]
    r = r.reshape(_BLK // 128, 128)
    gid = i * _BLK + jax.lax.broadcasted_iota(jnp.int32, (_BLK // 128, 128), 0) * 128 \
        + jax.lax.broadcasted_iota(jnp.int32, (_BLK // 128, 128), 1)
    o_ref[...] = jnp.where(gid < N, r, 0.0)


def _mlp(x, W1, b1, W2, b2):
    return pl.pallas_call(
        _mlp_body,
        grid=(_GRID,),
        in_specs=[
            pl.BlockSpec((_BLK, D), lambda i: (i, 0)),
            pl.BlockSpec((D, H), lambda i: (0, 0)),
            pl.BlockSpec((1, H), lambda i: (0, 0)),
            pl.BlockSpec((1, H), lambda i: (0, 0)),
            pl.BlockSpec((1, 1), lambda i: (0, 0)),
        ],
        out_specs=pl.BlockSpec((_BLK // 128, 128), lambda i: (i, 0)),
        out_shape=jax.ShapeDtypeStruct((_OROWS, 128), jnp.float32),
    )(x, W1, b1.reshape(1, H), W2.reshape(1, H), b2.reshape(1, 1))


_NS = 16
_ROWS = 896
_NPAD = _ROWS * 128
_RPW = _ROWS // _NS
_CHUNK = 28
_SPREAD = 8
_PLANE = G + 16
_ACC = _SPREAD * _PLANE
_GPW = G // _NS


def _segsum_body(vals_hbm, idx_hbm, out_hbm, vals_v, idx_v, stage_v, fold_v,
                 out_v, acc_sp, in_sem, sc_sem):
    s = lax.axis_index("s")
    base = s * _RPW
    vals_cp = pltpu.async_copy(vals_hbm.at[pl.ds(base, _RPW)], vals_v, in_sem)
    idx_cp = pltpu.async_copy(idx_hbm.at[pl.ds(base, _RPW)], idx_v, in_sem)

    @pl.when(s == 0)
    def _():
        for j in range(_ACC // 16):
            stage_v[pl.ds(j * 16, 16)] = jnp.zeros((16,), jnp.float32)
        pltpu.sync_copy(stage_v, acc_sp)

    vals_cp.wait()
    idx_cp.wait()
    plsc.subcore_barrier()

    def chunk(k, carry):
        cps = [
            pltpu.async_copy(
                vals_v.at[k * _CHUNK + j],
                acc_sp.at[idx_v.at[k * _CHUNK + j]],
                sc_sem,
                add=True,
            )
            for j in range(_CHUNK)
        ]
        for cp in cps:
            cp.wait()
        return carry

    lax.fori_loop(0, _RPW // _CHUNK, chunk, 0)
    plsc.subcore_barrier()

    fold_cps = [
        pltpu.async_copy(
            acc_sp.at[pl.ds(j * _PLANE + s * _GPW, _GPW)],
            fold_v.at[pl.ds(j * _GPW, _GPW)],
            in_sem,
        )
        for j in range(_SPREAD)
    ]
    for cp in fold_cps:
        cp.wait()
    for gb in range(_GPW // 16):
        tot = jnp.zeros((16,), jnp.float32)
        for j in range(_SPREAD):
            tot = tot + fold_v[pl.ds(j * _GPW + gb * 16, 16)]
        out_v[pl.ds(gb * 16, 16)] = tot
    pltpu.sync_copy(out_v, out_hbm.at[pl.ds(s * _GPW, _GPW)])


@functools.cache
def _make_segsum():
    return pl.kernel(
        _segsum_body,
        out_type=jax.ShapeDtypeStruct((G,), jnp.float32),
        mesh=plsc.VectorSubcoreMesh(
            core_axis_name="c", subcore_axis_name="s",
            num_cores=1, num_subcores=_NS,
        ),
        scratch_types=[
            pltpu.VMEM((_RPW, 128), jnp.float32),
            pltpu.VMEM((_RPW, 128), jnp.int32),
            pltpu.VMEM((_ACC,), jnp.float32),
            pltpu.VMEM((_SPREAD * _GPW,), jnp.float32),
            pltpu.VMEM((_GPW,), jnp.float32),
            pltpu.VMEM_SHARED((_ACC,), jnp.float32),
            pltpu.SemaphoreType.DMA,
            pltpu.SemaphoreType.DMA,
        ],
    )


def kernel(x_scalar, x_spherical, batch, W1, b1, W2, b2):
    res = _mlp(x_scalar, W1, b1, W2, b2)
    vals = jnp.pad(res, ((0, _ROWS - _OROWS), (0, 0)))
    spread = jnp.arange(N, dtype=jnp.int32) % _SPREAD
    idx = jnp.pad(batch + spread * _PLANE, (0, _NPAD - N))
    pad_pos = jnp.arange(_NPAD, dtype=jnp.int32)
    idx = jnp.where(pad_pos < N, idx, pad_pos % _ACC).reshape(_ROWS, 128)
    out = _make_segsum()(vals, idx)
    return out.reshape(G, 1)
